# initial kernel scaffold (unmeasured)
import jax
import jax.numpy as jnp
from jax import lax
from jax.experimental import pallas as pl
from jax.experimental.pallas import tpu as pltpu

N_DEV = 4
COMM_DTYPE = jnp.bfloat16


def kernel(x, w_mat, scale_x, scale_w):
    m_per, k = x.shape
    _, n = w_mat.shape
    n_per = n // N_DEV

    def body(x_ref, w_ref, sx_ref, sw_ref, out_ref,
             send_buf, recv_buf, send_sems, recv_sems):
        my = lax.axis_index("i")
        s = sx_ref[0] * sw_ref[0]

        barrier_sem = pltpu.get_barrier_semaphore()
        for off in range(1, N_DEV):
            pl.semaphore_signal(
                barrier_sem, inc=1,
                device_id=((my + off) % N_DEV,),
                device_id_type=pl.DeviceIdType.MESH,
            )
        pl.semaphore_wait(barrier_sem, N_DEV - 1)

        acc = lax.dot_general(
            x_ref[...], w_ref[...],
            (((1,), (0,)), ((), ())),
            preferred_element_type=jnp.float32,
        )
        y = jnp.maximum(acc * s, 0.0)

        sends = []
        for off in range(1, N_DEV):
            tgt = (my + off) % N_DEV
            blk = lax.dynamic_slice(y, (0, tgt * n_per), (m_per, n_per))
            send_buf[off - 1, :, :] = blk.astype(COMM_DTYPE)
            rdma = pltpu.make_async_remote_copy(
                src_ref=send_buf.at[off - 1],
                dst_ref=recv_buf.at[my],
                send_sem=send_sems.at[off - 1],
                recv_sem=recv_sems.at[my],
                device_id=(tgt,),
                device_id_type=pl.DeviceIdType.MESH,
            )
            rdma.start()
            sends.append(rdma)

        own = lax.dynamic_slice(y, (0, my * n_per), (m_per, n_per))
        out_ref[pl.ds(my * m_per, m_per), :] = own

        for off in range(1, N_DEV):
            src = (my - off) % N_DEV
            recv = pltpu.make_async_remote_copy(
                src_ref=send_buf.at[0],
                dst_ref=recv_buf.at[src],
                send_sem=send_sems.at[0],
                recv_sem=recv_sems.at[src],
                device_id=(src,),
                device_id_type=pl.DeviceIdType.MESH,
            )
            recv.wait_recv()
            out_ref[pl.ds(src * m_per, m_per), :] = (
                recv_buf[src, :, :].astype(jnp.float32))

        for rdma in sends:
            rdma.wait_send()

    return pl.pallas_call(
        body,
        out_shape=jax.ShapeDtypeStruct((N_DEV * m_per, n_per), jnp.float32),
        in_specs=[
            pl.BlockSpec(memory_space=pltpu.VMEM),
            pl.BlockSpec(memory_space=pltpu.VMEM),
            pl.BlockSpec(memory_space=pltpu.SMEM),
            pl.BlockSpec(memory_space=pltpu.SMEM),
        ],
        out_specs=pl.BlockSpec(memory_space=pltpu.VMEM),
        scratch_shapes=[
            pltpu.VMEM((N_DEV - 1, m_per, n_per), COMM_DTYPE),
            pltpu.VMEM((N_DEV, m_per, n_per), COMM_DTYPE),
            pltpu.SemaphoreType.DMA((N_DEV - 1,)),
            pltpu.SemaphoreType.DMA((N_DEV,)),
        ],
        compiler_params=pltpu.CompilerParams(collective_id=0),
    )(x, w_mat, scale_x, scale_w)


# baseline (device time: 63064 ns/iter reference)
import jax
import jax.numpy as jnp
from jax import lax
from jax.experimental import pallas as pl
from jax.experimental.pallas import tpu as pltpu

N_DEV = 4
COMM_DTYPE = jnp.bfloat16


def kernel(x, w_mat, scale_x, scale_w):
    m_per, k = x.shape
    _, n = w_mat.shape
    n_per = n // N_DEV

    def body(x_ref, w_ref, sx_ref, sw_ref, out_ref,
             send_buf, recv_buf, send_sems, recv_sems):
        b = pl.program_id(0)
        my = lax.axis_index("i")
        s = sx_ref[0] * sw_ref[0]

        def send_rdma(blk):
            return pltpu.make_async_remote_copy(
                src_ref=send_buf.at[blk],
                dst_ref=recv_buf.at[my],
                send_sem=send_sems.at[blk],
                recv_sem=recv_sems.at[my],
                device_id=(blk,),
                device_id_type=pl.DeviceIdType.MESH,
            )

        @pl.when(b == 0)
        def _():
            barrier_sem = pltpu.get_barrier_semaphore()
            for off in range(1, N_DEV):
                pl.semaphore_signal(
                    barrier_sem, inc=1,
                    device_id=((my + off) % N_DEV,),
                    device_id_type=pl.DeviceIdType.MESH,
                )
            pl.semaphore_wait(barrier_sem, N_DEV - 1)

        acc = lax.dot_general(
            x_ref[...].astype(COMM_DTYPE), w_ref[...].astype(COMM_DTYPE),
            (((1,), (0,)), ((), ())),
            preferred_element_type=jnp.float32,
        )
        yb = jnp.maximum(acc * s, 0.0)

        @pl.when(b == my)
        def _():
            out_ref[pl.ds(my * m_per, m_per), :] = yb

        @pl.when(b != my)
        def _():
            send_buf[b, :, :] = yb.astype(COMM_DTYPE)
            send_rdma(b).start()

        @pl.when(b == N_DEV - 1)
        def _():
            for src in range(N_DEV):
                @pl.when(src != my)
                def _():
                    recv = pltpu.make_async_remote_copy(
                        src_ref=send_buf.at[src],
                        dst_ref=recv_buf.at[src],
                        send_sem=send_sems.at[src],
                        recv_sem=recv_sems.at[src],
                        device_id=(src,),
                        device_id_type=pl.DeviceIdType.MESH,
                    )
                    recv.wait_recv()
                    out_ref[src * m_per:(src + 1) * m_per, :] = (
                        recv_buf[src, :, :].astype(jnp.float32))

            for blk in range(N_DEV):
                @pl.when(blk != my)
                def _():
                    send_rdma(blk).wait_send()

    return pl.pallas_call(
        body,
        grid=(N_DEV,),
        out_shape=jax.ShapeDtypeStruct((N_DEV * m_per, n_per), jnp.float32),
        in_specs=[
            pl.BlockSpec((m_per, k), lambda b: (0, 0)),
            pl.BlockSpec((k, n_per), lambda b: (0, b)),
            pl.BlockSpec(memory_space=pltpu.SMEM),
            pl.BlockSpec(memory_space=pltpu.SMEM),
        ],
        out_specs=pl.BlockSpec((N_DEV * m_per, n_per), lambda b: (0, 0)),
        scratch_shapes=[
            pltpu.VMEM((N_DEV, m_per, n_per), COMM_DTYPE),
            pltpu.VMEM((N_DEV, m_per, n_per), COMM_DTYPE),
            pltpu.SemaphoreType.DMA((N_DEV,)),
            pltpu.SemaphoreType.DMA((N_DEV,)),
        ],
        compiler_params=pltpu.CompilerParams(
            collective_id=0,
            dimension_semantics=("arbitrary",),
            vmem_limit_bytes=60 * 1024 * 1024,
        ),
    )(x, w_mat, scale_x, scale_w)


# device time: 55762 ns/iter; 1.1309x vs baseline; 1.1309x over previous
import jax
import jax.numpy as jnp
from jax import lax
from jax.experimental import pallas as pl
from jax.experimental.pallas import tpu as pltpu

N_DEV = 4
COMM_DTYPE = jnp.bfloat16
MXU_DTYPE = jnp.float8_e4m3fn


def kernel(x, w_mat, scale_x, scale_w):
    m_per, k = x.shape
    _, n = w_mat.shape
    n_per = n // N_DEV

    def body(x_ref, w_ref, sx_ref, sw_ref, out_ref,
             xb_buf, send_buf, recv_buf, send_sems, recv_sems):
        b = pl.program_id(0)
        my = lax.axis_index("i")
        s = sx_ref[0] * sw_ref[0]

        def send_rdma(blk):
            return pltpu.make_async_remote_copy(
                src_ref=send_buf.at[blk],
                dst_ref=recv_buf.at[my],
                send_sem=send_sems.at[blk],
                recv_sem=recv_sems.at[my],
                device_id=(blk,),
                device_id_type=pl.DeviceIdType.MESH,
            )

        @pl.when(b == 0)
        def _():
            barrier_sem = pltpu.get_barrier_semaphore()
            for off in range(1, N_DEV):
                pl.semaphore_signal(
                    barrier_sem, inc=1,
                    device_id=((my + off) % N_DEV,),
                    device_id_type=pl.DeviceIdType.MESH,
                )
            pl.semaphore_wait(barrier_sem, N_DEV - 1)

        @pl.when(b == 0)
        def _():
            xb_buf[...] = x_ref[...].astype(MXU_DTYPE)

        acc = lax.dot_general(
            xb_buf[...], w_ref[...].astype(MXU_DTYPE),
            (((1,), (0,)), ((), ())),
            preferred_element_type=jnp.float32,
        )
        yb = jnp.maximum(acc * s, 0.0)

        @pl.when(b == my)
        def _():
            out_ref[pl.ds(my * m_per, m_per), :] = yb

        @pl.when(b != my)
        def _():
            send_buf[b, :, :] = yb.astype(COMM_DTYPE)
            send_rdma(b).start()

        @pl.when(b == N_DEV - 1)
        def _():
            for src in range(N_DEV):
                @pl.when(src != my)
                def _():
                    recv = pltpu.make_async_remote_copy(
                        src_ref=send_buf.at[src],
                        dst_ref=recv_buf.at[src],
                        send_sem=send_sems.at[src],
                        recv_sem=recv_sems.at[src],
                        device_id=(src,),
                        device_id_type=pl.DeviceIdType.MESH,
                    )
                    recv.wait_recv()
                    out_ref[src * m_per:(src + 1) * m_per, :] = (
                        recv_buf[src, :, :].astype(jnp.float32))

            for blk in range(N_DEV):
                @pl.when(blk != my)
                def _():
                    send_rdma(blk).wait_send()

    return pl.pallas_call(
        body,
        grid=(N_DEV,),
        out_shape=jax.ShapeDtypeStruct((N_DEV * m_per, n_per), jnp.float32),
        in_specs=[
            pl.BlockSpec((m_per, k), lambda b: (0, 0)),
            pl.BlockSpec((k, n_per), lambda b: (0, b)),
            pl.BlockSpec(memory_space=pltpu.SMEM),
            pl.BlockSpec(memory_space=pltpu.SMEM),
        ],
        out_specs=pl.BlockSpec((N_DEV * m_per, n_per), lambda b: (0, 0)),
        scratch_shapes=[
            pltpu.VMEM((m_per, k), MXU_DTYPE),
            pltpu.VMEM((N_DEV, m_per, n_per), COMM_DTYPE),
            pltpu.VMEM((N_DEV, m_per, n_per), COMM_DTYPE),
            pltpu.SemaphoreType.DMA((N_DEV,)),
            pltpu.SemaphoreType.DMA((N_DEV,)),
        ],
        compiler_params=pltpu.CompilerParams(
            collective_id=0,
            dimension_semantics=("arbitrary",),
            vmem_limit_bytes=60 * 1024 * 1024,
        ),
    )(x, w_mat, scale_x, scale_w)


# device time: 48815 ns/iter; 1.2919x vs baseline; 1.1423x over previous
import jax
import jax.numpy as jnp
from jax import lax
from jax.experimental import pallas as pl
from jax.experimental.pallas import tpu as pltpu

N_DEV = 4
N_HALVES = 2
N_STEPS = N_DEV * N_HALVES
COMM_DTYPE = jnp.bfloat16
MXU_DTYPE = jnp.float8_e4m3fn

_OFFSETS = (2, 2, 1, 1, 3, 3, 0, 0)


def kernel(x, w_mat, scale_x, scale_w):
    m_per, k = x.shape
    _, n = w_mat.shape
    n_per = n // N_DEV
    n_half = n_per // N_HALVES

    my_out = lax.axis_index("i").astype(jnp.int32)
    tgt = (jnp.asarray(_OFFSETS, jnp.int32) + my_out) % N_DEV
    halves = jnp.arange(N_STEPS, dtype=jnp.int32) % N_HALVES
    blocks = tgt * N_HALVES + halves

    def body(blk_ref, x_ref, w_ref, sx_ref, sw_ref, out_ref,
             xb_buf, send_buf, recv_buf, send_sems, recv_sems):
        t = pl.program_id(0)
        my = lax.axis_index("i")
        s = sx_ref[0] * sw_ref[0]

        @pl.when(t == 0)
        def _():
            barrier_sem = pltpu.get_barrier_semaphore()
            for off in range(1, N_DEV):
                pl.semaphore_signal(
                    barrier_sem, inc=1,
                    device_id=((my + off) % N_DEV,),
                    device_id_type=pl.DeviceIdType.MESH,
                )
            pl.semaphore_wait(barrier_sem, N_DEV - 1)
            xb_buf[...] = x_ref[...].astype(MXU_DTYPE)

        acc = lax.dot_general(
            xb_buf[...], w_ref[...].astype(MXU_DTYPE),
            (((1,), (0,)), ((), ())),
            preferred_element_type=jnp.float32,
        )
        yb = jnp.maximum(acc * s, 0.0)

        @pl.when(t < N_STEPS - N_HALVES)
        def _():
            send_buf[t, :, :] = yb.astype(COMM_DTYPE)
            pltpu.make_async_remote_copy(
                src_ref=send_buf.at[t],
                dst_ref=recv_buf.at[my * N_HALVES + t % N_HALVES],
                send_sem=send_sems.at[t],
                recv_sem=recv_sems.at[my * N_HALVES + t % N_HALVES],
                device_id=(blk_ref[t] // N_HALVES,),
                device_id_type=pl.DeviceIdType.MESH,
            ).start()

        @pl.when(t >= N_STEPS - N_HALVES)
        def _():
            h = t % N_HALVES
            out_ref[pl.ds(my * m_per, m_per), pl.ds(h * n_half, n_half)] = yb

        @pl.when(t == N_STEPS - 1)
        def _():
            for src in range(N_DEV):
                for h in range(N_HALVES):
                    slot = src * N_HALVES + h

                    @pl.when(src != my)
                    def _():
                        recv = pltpu.make_async_remote_copy(
                            src_ref=send_buf.at[0],
                            dst_ref=recv_buf.at[slot],
                            send_sem=send_sems.at[0],
                            recv_sem=recv_sems.at[slot],
                            device_id=(src,),
                            device_id_type=pl.DeviceIdType.MESH,
                        )
                        recv.wait_recv()
                        out_ref[src * m_per:(src + 1) * m_per,
                                h * n_half:(h + 1) * n_half] = (
                            recv_buf[slot, :, :].astype(jnp.float32))

            for slot in range(N_STEPS - N_HALVES):
                pltpu.make_async_remote_copy(
                    src_ref=send_buf.at[slot],
                    dst_ref=recv_buf.at[0],
                    send_sem=send_sems.at[slot],
                    recv_sem=recv_sems.at[0],
                    device_id=(my,),
                    device_id_type=pl.DeviceIdType.MESH,
                ).wait_send()

    grid_spec = pltpu.PrefetchScalarGridSpec(
        num_scalar_prefetch=1,
        grid=(N_STEPS,),
        in_specs=[
            pl.BlockSpec((m_per, k), lambda t, blk: (0, 0)),
            pl.BlockSpec((k, n_half), lambda t, blk: (0, blk[t])),
            pl.BlockSpec(memory_space=pltpu.SMEM),
            pl.BlockSpec(memory_space=pltpu.SMEM),
        ],
        out_specs=pl.BlockSpec((N_DEV * m_per, n_per), lambda t, blk: (0, 0)),
        scratch_shapes=[
            pltpu.VMEM((m_per, k), MXU_DTYPE),
            pltpu.VMEM((N_STEPS - N_HALVES, m_per, n_half), COMM_DTYPE),
            pltpu.VMEM((N_DEV * N_HALVES, m_per, n_half), COMM_DTYPE),
            pltpu.SemaphoreType.DMA((N_STEPS - N_HALVES,)),
            pltpu.SemaphoreType.DMA((N_DEV * N_HALVES,)),
        ],
    )

    return pl.pallas_call(
        body,
        grid_spec=grid_spec,
        out_shape=jax.ShapeDtypeStruct((N_DEV * m_per, n_per), jnp.float32),
        compiler_params=pltpu.CompilerParams(
            collective_id=0,
            dimension_semantics=("arbitrary",),
            vmem_limit_bytes=60 * 1024 * 1024,
        ),
    )(blocks, x, w_mat, scale_x, scale_w)


# device time: 48742 ns/iter; 1.2938x vs baseline; 1.0015x over previous
import jax
import jax.numpy as jnp
from jax import lax
from jax.experimental import pallas as pl
from jax.experimental.pallas import tpu as pltpu

N_DEV = 4
N_HALVES = 2
N_STEPS = N_DEV * N_HALVES
COMM_DTYPE = jnp.bfloat16
MXU_DTYPE = jnp.float8_e4m3fn

_OFFSETS = (2, 1, 3, 2, 1, 3, 0, 0)


def kernel(x, w_mat, scale_x, scale_w):
    m_per, k = x.shape
    _, n = w_mat.shape
    n_per = n // N_DEV
    n_half = n_per // N_HALVES

    my_out = lax.axis_index("i").astype(jnp.int32)
    tgt = (jnp.asarray(_OFFSETS, jnp.int32) + my_out) % N_DEV
    halves = jnp.arange(N_STEPS, dtype=jnp.int32) % N_HALVES
    blocks = tgt * N_HALVES + halves

    def body(blk_ref, x_ref, w_ref, sx_ref, sw_ref, out_ref,
             xb_buf, send_buf, recv_buf, send_sems, recv_sems):
        t = pl.program_id(0)
        my = lax.axis_index("i")
        s = sx_ref[0] * sw_ref[0]

        @pl.when(t == 0)
        def _():
            barrier_sem = pltpu.get_barrier_semaphore()
            for off in range(1, N_DEV):
                pl.semaphore_signal(
                    barrier_sem, inc=1,
                    device_id=((my + off) % N_DEV,),
                    device_id_type=pl.DeviceIdType.MESH,
                )
            pl.semaphore_wait(barrier_sem, N_DEV - 1)
            xb_buf[...] = x_ref[...].astype(MXU_DTYPE)

        acc = lax.dot_general(
            xb_buf[...], w_ref[...].astype(MXU_DTYPE),
            (((1,), (0,)), ((), ())),
            preferred_element_type=jnp.float32,
        )
        yb = jnp.maximum(acc * s, 0.0)

        @pl.when(t < N_STEPS - N_HALVES)
        def _():
            send_buf[t, :, :] = yb.astype(COMM_DTYPE)
            pltpu.make_async_remote_copy(
                src_ref=send_buf.at[t],
                dst_ref=recv_buf.at[my * N_HALVES + t % N_HALVES],
                send_sem=send_sems.at[t],
                recv_sem=recv_sems.at[my * N_HALVES + t % N_HALVES],
                device_id=(blk_ref[t] // N_HALVES,),
                device_id_type=pl.DeviceIdType.MESH,
            ).start()

        @pl.when(t >= N_STEPS - N_HALVES)
        def _():
            h = t % N_HALVES
            out_ref[pl.ds(my * m_per, m_per), pl.ds(h * n_half, n_half)] = yb

        @pl.when(t == N_STEPS - 1)
        def _():
            for src in range(N_DEV):
                for h in range(N_HALVES):
                    slot = src * N_HALVES + h

                    @pl.when(src != my)
                    def _():
                        recv = pltpu.make_async_remote_copy(
                            src_ref=send_buf.at[0],
                            dst_ref=recv_buf.at[slot],
                            send_sem=send_sems.at[0],
                            recv_sem=recv_sems.at[slot],
                            device_id=(src,),
                            device_id_type=pl.DeviceIdType.MESH,
                        )
                        recv.wait_recv()
                        out_ref[src * m_per:(src + 1) * m_per,
                                h * n_half:(h + 1) * n_half] = (
                            recv_buf[slot, :, :].astype(jnp.float32))

            for slot in range(N_STEPS - N_HALVES):
                pltpu.make_async_remote_copy(
                    src_ref=send_buf.at[slot],
                    dst_ref=recv_buf.at[0],
                    send_sem=send_sems.at[slot],
                    recv_sem=recv_sems.at[0],
                    device_id=(my,),
                    device_id_type=pl.DeviceIdType.MESH,
                ).wait_send()

    grid_spec = pltpu.PrefetchScalarGridSpec(
        num_scalar_prefetch=1,
        grid=(N_STEPS,),
        in_specs=[
            pl.BlockSpec((m_per, k), lambda t, blk: (0, 0)),
            pl.BlockSpec((k, n_half), lambda t, blk: (0, blk[t])),
            pl.BlockSpec(memory_space=pltpu.SMEM),
            pl.BlockSpec(memory_space=pltpu.SMEM),
        ],
        out_specs=pl.BlockSpec((N_DEV * m_per, n_per), lambda t, blk: (0, 0)),
        scratch_shapes=[
            pltpu.VMEM((m_per, k), MXU_DTYPE),
            pltpu.VMEM((N_STEPS - N_HALVES, m_per, n_half), COMM_DTYPE),
            pltpu.VMEM((N_DEV * N_HALVES, m_per, n_half), COMM_DTYPE),
            pltpu.SemaphoreType.DMA((N_STEPS - N_HALVES,)),
            pltpu.SemaphoreType.DMA((N_DEV * N_HALVES,)),
        ],
    )

    return pl.pallas_call(
        body,
        grid_spec=grid_spec,
        out_shape=jax.ShapeDtypeStruct((N_DEV * m_per, n_per), jnp.float32),
        compiler_params=pltpu.CompilerParams(
            collective_id=0,
            dimension_semantics=("arbitrary",),
            vmem_limit_bytes=60 * 1024 * 1024,
        ),
    )(blocks, x, w_mat, scale_x, scale_w)
